# initial kernel scaffold (unmeasured)
import functools

import jax
import jax.numpy as jnp
from jax import lax
from jax.experimental import pallas as pl
from jax.experimental.pallas import tpu as pltpu

N_DEV = 8


def kernel(x, w_mat):
    m, k_shard = x.shape
    _, n = w_mat.shape
    m_chunk = m // N_DEV

    def body(x_ref, w_ref, out_ref, buf_ref, send_sems, recv_sems):
        my = lax.axis_index("i")
        left = (my - 1) % N_DEV
        right = (my + 1) % N_DEV

        barrier_sem = pltpu.get_barrier_semaphore()
        for nbr in (left, right):
            pl.semaphore_signal(
                barrier_sem, inc=1,
                device_id=(nbr,), device_id_type=pl.DeviceIdType.MESH,
            )
        pl.semaphore_wait(barrier_sem, 2)

        def partial_chunk(c):
            rows = x_ref[pl.ds(c * m_chunk, m_chunk), :]
            return jnp.dot(rows, w_ref[:, :], preferred_element_type=jnp.float32)

        c0 = (my + N_DEV - 1) % N_DEV
        buf_ref[0, :, :] = partial_chunk(c0)

        for s in range(N_DEV - 1):
            rdma = pltpu.make_async_remote_copy(
                src_ref=buf_ref.at[s],
                dst_ref=buf_ref.at[s + 1],
                send_sem=send_sems.at[s],
                recv_sem=recv_sems.at[s],
                device_id=(right,),
                device_id_type=pl.DeviceIdType.MESH,
            )
            rdma.start()
            rdma.wait()
            c = (my + N_DEV - 2 - s) % N_DEV
            buf_ref[s + 1, :, :] = buf_ref[s + 1, :, :] + partial_chunk(c)

        acc = buf_ref[N_DEV - 1, :, :]
        out_ref[:, :] = acc * (1.0 / (1.0 + jnp.exp(-acc)))

        @functools.partial(
            pl.run_scoped, second_barrier=pltpu.SemaphoreType.REGULAR
        )
        def _(second_barrier):
            for nbr in (left, right):
                pl.semaphore_signal(
                    second_barrier, inc=1,
                    device_id=(nbr,), device_id_type=pl.DeviceIdType.MESH,
                )
            pl.semaphore_wait(second_barrier, 2)

    return pl.pallas_call(
        body,
        out_shape=jax.ShapeDtypeStruct((m_chunk, n), jnp.float32),
        in_specs=[
            pl.BlockSpec(memory_space=pltpu.VMEM),
            pl.BlockSpec(memory_space=pltpu.VMEM),
        ],
        out_specs=pl.BlockSpec(memory_space=pltpu.VMEM),
        scratch_shapes=[
            pltpu.VMEM((N_DEV, m_chunk, n), jnp.float32),
            pltpu.SemaphoreType.DMA((N_DEV - 1,)),
            pltpu.SemaphoreType.DMA((N_DEV - 1,)),
        ],
        compiler_params=pltpu.CompilerParams(collective_id=0),
    )(x, w_mat)


# baseline (device time: 352220 ns/iter reference)
import functools

import jax
import jax.numpy as jnp
from jax import lax
from jax.experimental import pallas as pl
from jax.experimental.pallas import tpu as pltpu

N_DEV = 8


def kernel(x, w_mat):
    m, k_shard = x.shape
    _, n = w_mat.shape
    m_chunk = m // N_DEV

    def body(x_ref, w_ref, out_ref, buf_ref, send_sems, recv_sems, credit_sem):
        my = lax.axis_index("i")
        left = (my - 1) % N_DEV
        right = (my + 1) % N_DEV

        barrier_sem = pltpu.get_barrier_semaphore()
        for nbr in (left, right):
            pl.semaphore_signal(
                barrier_sem, inc=1,
                device_id=(nbr,), device_id_type=pl.DeviceIdType.MESH,
            )
        pl.semaphore_wait(barrier_sem, 2)

        def partial_chunk(c):
            rows = x_ref[pl.ds(c * m_chunk, m_chunk), :]
            return jnp.dot(rows, w_ref[:, :], preferred_element_type=jnp.float32)

        c0 = (my + N_DEV - 1) % N_DEV
        buf_ref[0, :, :] = partial_chunk(c0)

        for s in range(N_DEV - 1):
            send_slot = s % 2
            recv_slot = (s + 1) % 2
            if s >= 1:
                pl.semaphore_wait(credit_sem, 1)
            rdma = pltpu.make_async_remote_copy(
                src_ref=buf_ref.at[send_slot],
                dst_ref=buf_ref.at[recv_slot],
                send_sem=send_sems.at[send_slot],
                recv_sem=recv_sems.at[recv_slot],
                device_id=(right,),
                device_id_type=pl.DeviceIdType.MESH,
            )
            rdma.start()
            rdma.wait()
            if s < N_DEV - 2:
                pl.semaphore_signal(
                    credit_sem, inc=1,
                    device_id=(left,), device_id_type=pl.DeviceIdType.MESH,
                )
            c = (my + N_DEV - 2 - s) % N_DEV
            buf_ref[recv_slot, :, :] = buf_ref[recv_slot, :, :] + partial_chunk(c)

        acc = buf_ref[(N_DEV - 1) % 2, :, :]
        out_ref[:, :] = acc * (1.0 / (1.0 + jnp.exp(-acc)))

        @functools.partial(
            pl.run_scoped, second_barrier=pltpu.SemaphoreType.REGULAR
        )
        def _(second_barrier):
            for nbr in (left, right):
                pl.semaphore_signal(
                    second_barrier, inc=1,
                    device_id=(nbr,), device_id_type=pl.DeviceIdType.MESH,
                )
            pl.semaphore_wait(second_barrier, 2)

    return pl.pallas_call(
        body,
        out_shape=jax.ShapeDtypeStruct((m_chunk, n), jnp.float32),
        in_specs=[
            pl.BlockSpec(memory_space=pltpu.VMEM),
            pl.BlockSpec(memory_space=pltpu.VMEM),
        ],
        out_specs=pl.BlockSpec(memory_space=pltpu.VMEM),
        scratch_shapes=[
            pltpu.VMEM((2, m_chunk, n), jnp.float32),
            pltpu.SemaphoreType.DMA((2,)),
            pltpu.SemaphoreType.DMA((2,)),
            pltpu.SemaphoreType.REGULAR,
        ],
        compiler_params=pltpu.CompilerParams(collective_id=0),
    )(x, w_mat)


# device time: 195280 ns/iter; 1.8037x vs baseline; 1.8037x over previous
import functools

import jax
import jax.numpy as jnp
from jax import lax
from jax.experimental import pallas as pl
from jax.experimental.pallas import tpu as pltpu

N_DEV = 8


def kernel(x, w_mat):
    m, k_shard = x.shape
    _, n = w_mat.shape
    m_chunk = m // N_DEV
    n2 = n // 2

    def body(x_ref, w_ref, out_ref,
             fbuf_ref, bbuf_ref,
             fsend_sems, frecv_sems, bsend_sems, brecv_sems,
             fcredit_sem, bcredit_sem):
        my = lax.axis_index("i")
        left = (my - 1) % N_DEV
        right = (my + 1) % N_DEV

        barrier_sem = pltpu.get_barrier_semaphore()
        for nbr in (left, right):
            pl.semaphore_signal(
                barrier_sem, inc=1,
                device_id=(nbr,), device_id_type=pl.DeviceIdType.MESH,
            )
        pl.semaphore_wait(barrier_sem, 2)

        def fchunk(c):
            rows = x_ref[pl.ds(c * m_chunk, m_chunk), :]
            return jnp.dot(rows, w_ref[:, :n2], preferred_element_type=jnp.float32)

        def bchunk(c):
            rows = x_ref[pl.ds(c * m_chunk, m_chunk), :]
            return jnp.dot(rows, w_ref[:, n2:], preferred_element_type=jnp.float32)

        fbuf_ref[0, :, :] = fchunk((my + N_DEV - 1) % N_DEV)
        bbuf_ref[0, :, :] = bchunk((my + 1) % N_DEV)

        for s in range(N_DEV - 1):
            send_slot = s % 2
            recv_slot = (s + 1) % 2
            if s >= 1:
                pl.semaphore_wait(fcredit_sem, 1)
                pl.semaphore_wait(bcredit_sem, 1)
            frdma = pltpu.make_async_remote_copy(
                src_ref=fbuf_ref.at[send_slot],
                dst_ref=fbuf_ref.at[recv_slot],
                send_sem=fsend_sems.at[send_slot],
                recv_sem=frecv_sems.at[recv_slot],
                device_id=(right,),
                device_id_type=pl.DeviceIdType.MESH,
            )
            brdma = pltpu.make_async_remote_copy(
                src_ref=bbuf_ref.at[send_slot],
                dst_ref=bbuf_ref.at[recv_slot],
                send_sem=bsend_sems.at[send_slot],
                recv_sem=brecv_sems.at[recv_slot],
                device_id=(left,),
                device_id_type=pl.DeviceIdType.MESH,
            )
            frdma.start()
            brdma.start()
            cf = (my + N_DEV - 2 - s) % N_DEV
            cb = (my + 2 + s) % N_DEV
            tf = fchunk(cf)
            tb = bchunk(cb)
            frdma.wait()
            brdma.wait()
            if s < N_DEV - 2:
                pl.semaphore_signal(
                    fcredit_sem, inc=1,
                    device_id=(left,), device_id_type=pl.DeviceIdType.MESH,
                )
                pl.semaphore_signal(
                    bcredit_sem, inc=1,
                    device_id=(right,), device_id_type=pl.DeviceIdType.MESH,
                )
            fbuf_ref[recv_slot, :, :] = fbuf_ref[recv_slot, :, :] + tf
            bbuf_ref[recv_slot, :, :] = bbuf_ref[recv_slot, :, :] + tb

        last = (N_DEV - 1) % 2
        accf = fbuf_ref[last, :, :]
        accb = bbuf_ref[last, :, :]
        out_ref[:, :n2] = accf * (1.0 / (1.0 + jnp.exp(-accf)))
        out_ref[:, n2:] = accb * (1.0 / (1.0 + jnp.exp(-accb)))

        @functools.partial(
            pl.run_scoped, second_barrier=pltpu.SemaphoreType.REGULAR
        )
        def _(second_barrier):
            for nbr in (left, right):
                pl.semaphore_signal(
                    second_barrier, inc=1,
                    device_id=(nbr,), device_id_type=pl.DeviceIdType.MESH,
                )
            pl.semaphore_wait(second_barrier, 2)

    return pl.pallas_call(
        body,
        out_shape=jax.ShapeDtypeStruct((m_chunk, n), jnp.float32),
        in_specs=[
            pl.BlockSpec(memory_space=pltpu.VMEM),
            pl.BlockSpec(memory_space=pltpu.VMEM),
        ],
        out_specs=pl.BlockSpec(memory_space=pltpu.VMEM),
        scratch_shapes=[
            pltpu.VMEM((2, m_chunk, n2), jnp.float32),
            pltpu.VMEM((2, m_chunk, n2), jnp.float32),
            pltpu.SemaphoreType.DMA((2,)),
            pltpu.SemaphoreType.DMA((2,)),
            pltpu.SemaphoreType.DMA((2,)),
            pltpu.SemaphoreType.DMA((2,)),
            pltpu.SemaphoreType.REGULAR,
            pltpu.SemaphoreType.REGULAR,
        ],
        compiler_params=pltpu.CompilerParams(collective_id=0),
    )(x, w_mat)


# device time: 186435 ns/iter; 1.8892x vs baseline; 1.0474x over previous
import functools

import jax
import jax.numpy as jnp
from jax import lax
from jax.experimental import pallas as pl
from jax.experimental.pallas import tpu as pltpu

N_DEV = 8
K_SLOT = 3


def kernel(x, w_mat):
    m, k_shard = x.shape
    _, n = w_mat.shape
    m_chunk = m // N_DEV
    n2 = n // 2

    def body(x_ref, w_ref, out_ref,
             fbuf_ref, bbuf_ref,
             fsend_sems, frecv_sems, bsend_sems, brecv_sems,
             fcredit_sem, bcredit_sem):
        my = lax.axis_index("i")
        left = (my - 1) % N_DEV
        right = (my + 1) % N_DEV

        barrier_sem = pltpu.get_barrier_semaphore()
        for nbr in (left, right):
            pl.semaphore_signal(
                barrier_sem, inc=1,
                device_id=(nbr,), device_id_type=pl.DeviceIdType.MESH,
            )

        def fchunk(c):
            rows = x_ref[pl.ds(c * m_chunk, m_chunk), :]
            return jnp.dot(rows, w_ref[:, :n2], preferred_element_type=jnp.float32)

        def bchunk(c):
            rows = x_ref[pl.ds(c * m_chunk, m_chunk), :]
            return jnp.dot(rows, w_ref[:, n2:], preferred_element_type=jnp.float32)

        fbuf_ref[0, :, :] = fchunk((my + N_DEV - 1) % N_DEV)
        bbuf_ref[0, :, :] = bchunk((my + 1) % N_DEV)

        pl.semaphore_wait(barrier_sem, 2)

        def make_f(s):
            return pltpu.make_async_remote_copy(
                src_ref=fbuf_ref.at[s % K_SLOT],
                dst_ref=fbuf_ref.at[(s + 1) % K_SLOT],
                send_sem=fsend_sems.at[s % K_SLOT],
                recv_sem=frecv_sems.at[(s + 1) % K_SLOT],
                device_id=(right,),
                device_id_type=pl.DeviceIdType.MESH,
            )

        def make_b(s):
            return pltpu.make_async_remote_copy(
                src_ref=bbuf_ref.at[s % K_SLOT],
                dst_ref=bbuf_ref.at[(s + 1) % K_SLOT],
                send_sem=bsend_sems.at[s % K_SLOT],
                recv_sem=brecv_sems.at[(s + 1) % K_SLOT],
                device_id=(left,),
                device_id_type=pl.DeviceIdType.MESH,
            )

        f_cur = make_f(0)
        b_cur = make_b(0)
        f_cur.start()
        b_cur.start()

        for s in range(N_DEV - 1):
            slot = (s + 1) % K_SLOT
            tf = fchunk((my + N_DEV - 2 - s) % N_DEV)
            tb = bchunk((my + 2 + s) % N_DEV)

            f_cur.wait()
            if s <= N_DEV - 4:
                pl.semaphore_signal(
                    fcredit_sem, inc=1,
                    device_id=(left,), device_id_type=pl.DeviceIdType.MESH,
                )
            fbuf_ref[slot, :, :] = fbuf_ref[slot, :, :] + tf
            if s < N_DEV - 2:
                if s + 1 >= 2:
                    pl.semaphore_wait(fcredit_sem, 1)
                f_nxt = make_f(s + 1)
                f_nxt.start()
                f_cur = f_nxt

            b_cur.wait()
            if s <= N_DEV - 4:
                pl.semaphore_signal(
                    bcredit_sem, inc=1,
                    device_id=(right,), device_id_type=pl.DeviceIdType.MESH,
                )
            bbuf_ref[slot, :, :] = bbuf_ref[slot, :, :] + tb
            if s < N_DEV - 2:
                if s + 1 >= 2:
                    pl.semaphore_wait(bcredit_sem, 1)
                b_nxt = make_b(s + 1)
                b_nxt.start()
                b_cur = b_nxt

        last = (N_DEV - 1) % K_SLOT
        accf = fbuf_ref[last, :, :]
        accb = bbuf_ref[last, :, :]
        out_ref[:, :n2] = accf * (1.0 / (1.0 + jnp.exp(-accf)))
        out_ref[:, n2:] = accb * (1.0 / (1.0 + jnp.exp(-accb)))

        @functools.partial(
            pl.run_scoped, second_barrier=pltpu.SemaphoreType.REGULAR
        )
        def _(second_barrier):
            for nbr in (left, right):
                pl.semaphore_signal(
                    second_barrier, inc=1,
                    device_id=(nbr,), device_id_type=pl.DeviceIdType.MESH,
                )
            pl.semaphore_wait(second_barrier, 2)

    return pl.pallas_call(
        body,
        out_shape=jax.ShapeDtypeStruct((m_chunk, n), jnp.float32),
        in_specs=[
            pl.BlockSpec(memory_space=pltpu.VMEM),
            pl.BlockSpec(memory_space=pltpu.VMEM),
        ],
        out_specs=pl.BlockSpec(memory_space=pltpu.VMEM),
        scratch_shapes=[
            pltpu.VMEM((K_SLOT, m_chunk, n2), jnp.float32),
            pltpu.VMEM((K_SLOT, m_chunk, n2), jnp.float32),
            pltpu.SemaphoreType.DMA((K_SLOT,)),
            pltpu.SemaphoreType.DMA((K_SLOT,)),
            pltpu.SemaphoreType.DMA((K_SLOT,)),
            pltpu.SemaphoreType.DMA((K_SLOT,)),
            pltpu.SemaphoreType.REGULAR,
            pltpu.SemaphoreType.REGULAR,
        ],
        compiler_params=pltpu.CompilerParams(collective_id=0),
    )(x, w_mat)


# device time: 173350 ns/iter; 2.0318x vs baseline; 1.0755x over previous
import functools

import jax
import jax.numpy as jnp
from jax import lax
from jax.experimental import pallas as pl
from jax.experimental.pallas import tpu as pltpu

N_DEV = 8
K_SLOT = 3
N_STREAM = 4


def kernel(x, w_mat):
    m, k_shard = x.shape
    _, n = w_mat.shape
    m_chunk = m // N_DEV
    nq = n // 4

    def body(x_ref, w_ref, out_ref,
             bufs, send_sems, recv_sems, credit_sems):
        my = lax.axis_index("i")
        left = (my - 1) % N_DEV
        right = (my + 1) % N_DEV

        barrier_sem = pltpu.get_barrier_semaphore()
        for nbr in (left, right):
            pl.semaphore_signal(
                barrier_sem, inc=1,
                device_id=(nbr,), device_id_type=pl.DeviceIdType.MESH,
            )

        def fchunk(c):
            rows = x_ref[pl.ds(c * m_chunk, m_chunk), :]
            return jnp.dot(rows, w_ref[:, : 2 * nq],
                           preferred_element_type=jnp.float32)

        def bchunk(c):
            rows = x_ref[pl.ds(c * m_chunk, m_chunk), :]
            return jnp.dot(rows, w_ref[:, 2 * nq:],
                           preferred_element_type=jnp.float32)

        STREAMS = (
            dict(i=0, dst=1, peer=-1, qcol=0),
            dict(i=2, dst=-1, peer=1, qcol=2),
            dict(i=1, dst=1, peer=-1, qcol=1),
            dict(i=3, dst=-1, peer=1, qcol=3),
        )

        def send_chunk(st):
            def _f(s):
                if st["dst"] == 1:
                    return (my + N_DEV - 1 - s) % N_DEV
                return (my + 1 + s) % N_DEV
            return _f

        t0f = fchunk((my + N_DEV - 1) % N_DEV)
        t0b = bchunk((my + 1) % N_DEV)
        init = {0: t0f[:, :nq], 1: t0f[:, nq:], 2: t0b[:, :nq], 3: t0b[:, nq:]}
        for st in STREAMS:
            bufs[st["i"]][0, :, :] = init[st["i"]]

        pl.semaphore_wait(barrier_sem, 2)

        def make_rdma(st, s):
            i = st["i"]
            dev = right if st["dst"] == 1 else left
            return pltpu.make_async_remote_copy(
                src_ref=bufs[i].at[s % K_SLOT],
                dst_ref=bufs[i].at[(s + 1) % K_SLOT],
                send_sem=send_sems.at[i, s % K_SLOT],
                recv_sem=recv_sems.at[i, (s + 1) % K_SLOT],
                device_id=(dev,),
                device_id_type=pl.DeviceIdType.MESH,
            )

        cur = {}
        for st in STREAMS:
            cur[st["i"]] = make_rdma(st, 0)
            cur[st["i"]].start()

        for s in range(N_DEV - 1):
            slot = (s + 1) % K_SLOT
            tf = fchunk((my + N_DEV - 2 - s) % N_DEV)
            tb = bchunk((my + 2 + s) % N_DEV)
            temps = {0: tf[:, :nq], 1: tf[:, nq:], 2: tb[:, :nq], 3: tb[:, nq:]}

            for st in STREAMS:
                i = st["i"]
                cur[i].wait()
                if s <= N_DEV - 4:
                    pl.semaphore_signal(
                        credit_sems.at[i], inc=1,
                        device_id=((my + st["peer"]) % N_DEV,),
                        device_id_type=pl.DeviceIdType.MESH,
                    )
                acc = bufs[i][slot, :, :] + temps[i]
                if s < N_DEV - 2:
                    bufs[i][slot, :, :] = acc
                    if s + 1 >= 2:
                        pl.semaphore_wait(credit_sems.at[i], 1)
                    cur[i] = make_rdma(st, s + 1)
                    cur[i].start()
                else:
                    q = st["qcol"]
                    out_ref[:, q * nq:(q + 1) * nq] = acc * (
                        1.0 / (1.0 + jnp.exp(-acc))
                    )

        @functools.partial(
            pl.run_scoped, second_barrier=pltpu.SemaphoreType.REGULAR
        )
        def _(second_barrier):
            for nbr in (left, right):
                pl.semaphore_signal(
                    second_barrier, inc=1,
                    device_id=(nbr,), device_id_type=pl.DeviceIdType.MESH,
                )
            pl.semaphore_wait(second_barrier, 2)

    def wrapped_body(x_ref, w_ref, out_ref,
                     buf0, buf1, buf2, buf3,
                     send_sems, recv_sems, credit_sems):
        body(x_ref, w_ref, out_ref, [buf0, buf1, buf2, buf3],
             send_sems, recv_sems, credit_sems)

    return pl.pallas_call(
        wrapped_body,
        out_shape=jax.ShapeDtypeStruct((m_chunk, n), jnp.float32),
        in_specs=[
            pl.BlockSpec(memory_space=pltpu.VMEM),
            pl.BlockSpec(memory_space=pltpu.VMEM),
        ],
        out_specs=pl.BlockSpec(memory_space=pltpu.VMEM),
        scratch_shapes=[
            pltpu.VMEM((K_SLOT, m_chunk, nq), jnp.float32),
            pltpu.VMEM((K_SLOT, m_chunk, nq), jnp.float32),
            pltpu.VMEM((K_SLOT, m_chunk, nq), jnp.float32),
            pltpu.VMEM((K_SLOT, m_chunk, nq), jnp.float32),
            pltpu.SemaphoreType.DMA((N_STREAM, K_SLOT)),
            pltpu.SemaphoreType.DMA((N_STREAM, K_SLOT)),
            pltpu.SemaphoreType.REGULAR((N_STREAM,)),
        ],
        compiler_params=pltpu.CompilerParams(collective_id=0),
    )(x, w_mat)
